# baseline (speedup 1.0000x reference)
import functools

import jax
import jax.numpy as jnp
from jax import lax
from jax.experimental import pallas as pl
from jax.experimental.pallas import tpu as pltpu
from jax.experimental.pallas import tpu_sc as plsc

_INFO = plsc.get_sparse_core_info()
_NC = _INFO.num_cores
_NS = _INFO.num_subcores
_NW = _NC * _NS


def _make_gather(n_rows: int, seq: int, hidden: int, vocab: int):
    assert n_rows % _NW == 0
    rows_per_w = n_rows // _NW
    assert rows_per_w >= 2 and rows_per_w % 2 == 0

    mesh = plsc.VectorSubcoreMesh(core_axis_name="c", subcore_axis_name="s")

    @functools.partial(
        pl.kernel,
        out_type=jax.ShapeDtypeStruct((n_rows, seq, hidden), jnp.float32),
        mesh=mesh,
        compiler_params=pltpu.CompilerParams(use_tc_tiling_on_sc=True),
        scratch_types=[
            pltpu.VMEM((rows_per_w, seq), jnp.int32),
            pltpu.VMEM((2, seq, hidden), jnp.float32),
            pltpu.SemaphoreType.DMA,
            pltpu.SemaphoreType.DMA,
            pltpu.SemaphoreType.DMA,
            pltpu.SemaphoreType.DMA,
        ],
    )
    def gather_kernel(table_hbm, idx_hbm, out_hbm, idx_v, rows_v, g0, g1, w0, w1):
        gsem = (g0, g1)
        wsem = (w0, w1)
        wid = lax.axis_index("s") * _NC + lax.axis_index("c")
        row_base = wid * rows_per_w
        pltpu.sync_copy(idx_hbm.at[pl.ds(row_base, rows_per_w)], idx_v)

        def start_gather(i, p):
            pltpu.async_copy(table_hbm.at[idx_v.at[i]], rows_v.at[p], gsem[p])

        def wait_gather(p):
            pltpu.make_async_copy(
                table_hbm.at[idx_v.at[0]], rows_v.at[p], gsem[p]
            ).wait()

        def start_wb(i, p):
            pltpu.async_copy(rows_v.at[p], out_hbm.at[row_base + i], wsem[p])

        def wait_wb(p):
            pltpu.make_async_copy(rows_v.at[p], out_hbm.at[0], wsem[p]).wait()

        start_gather(0, 0)
        wait_gather(0)
        start_wb(0, 0)
        start_gather(1, 1)

        @pl.loop(1, rows_per_w - 1, step=2)
        def _body(j):
            for b in range(2):
                i = j + b
                p = (1 + b) % 2
                pn = 1 - p
                wait_gather(p)
                start_wb(i, p)
                wait_wb(pn)
                start_gather(i + 1, pn)

        wait_gather(1)
        start_wb(rows_per_w - 1, 1)
        wait_wb(0)
        wait_wb(1)

    return gather_kernel


@jax.jit
def kernel(x, weight):
    b, s = x.shape
    vocab, hidden = weight.shape
    return _make_gather(b, s, hidden, vocab)(weight, x.astype(jnp.int32))


# speedup vs baseline: 1.2160x; 1.2160x over previous
import functools

import jax
import jax.numpy as jnp
from jax import lax
from jax.experimental import pallas as pl
from jax.experimental.pallas import tpu as pltpu
from jax.experimental.pallas import tpu_sc as plsc

_INFO = plsc.get_sparse_core_info()
_NC = _INFO.num_cores
_NS = _INFO.num_subcores
_NW = _NC * _NS

_G = 2


def _make_gather(n_rows: int, seq: int, hidden: int, vocab: int):
    assert n_rows % (_NW * _G) == 0
    assert _G * seq <= 128
    chunks = n_rows // (_NW * _G)
    assert chunks >= 2 and chunks % 2 == 0
    gseq = _G * seq

    mesh = plsc.VectorSubcoreMesh(core_axis_name="c", subcore_axis_name="s")

    @functools.partial(
        pl.kernel,
        out_type=jax.ShapeDtypeStruct((n_rows, seq, hidden), jnp.float32),
        mesh=mesh,
        scratch_types=[
            pltpu.VMEM((chunks, gseq), jnp.int32),
            pltpu.VMEM((2, gseq, hidden), jnp.float32),
            pltpu.SemaphoreType.DMA,
            pltpu.SemaphoreType.DMA,
            pltpu.SemaphoreType.DMA,
            pltpu.SemaphoreType.DMA,
        ],
    )
    def gather_kernel(table_hbm, idx_hbm, out_hbm, idx_v, rows_v, g0, g1, w0, w1):
        gsem = (g0, g1)
        wsem = (w0, w1)
        wid = lax.axis_index("s") * _NC + lax.axis_index("c")
        row_base = wid * chunks * _G
        pltpu.sync_copy(idx_hbm.at[wid], idx_v)

        def start_gather(i, p):
            pltpu.async_copy(table_hbm.at[idx_v.at[i]], rows_v.at[p], gsem[p])

        def wait_gather(p):
            pltpu.make_async_copy(
                table_hbm.at[idx_v.at[0]], rows_v.at[p], gsem[p]
            ).wait()

        def start_wb(i, p):
            r = row_base + i * _G
            for g in range(_G):
                pltpu.async_copy(
                    rows_v.at[p, pl.ds(g * seq, seq)], out_hbm.at[r + g], wsem[p]
                )

        def wait_wb(p):
            for _ in range(_G):
                pltpu.make_async_copy(
                    rows_v.at[p, pl.ds(0, seq)], out_hbm.at[0], wsem[p]
                ).wait()

        start_gather(0, 0)
        wait_gather(0)
        start_wb(0, 0)
        start_gather(1, 1)

        @pl.loop(1, chunks - 1, step=2)
        def _body(j):
            for b in range(2):
                i = j + b
                p = (1 + b) % 2
                pn = 1 - p
                wait_gather(p)
                start_wb(i, p)
                wait_wb(pn)
                start_gather(i + 1, pn)

        wait_gather(1)
        start_wb(chunks - 1, 1)
        wait_wb(0)
        wait_wb(1)

    return gather_kernel


@jax.jit
def kernel(x, weight):
    b, s = x.shape
    vocab, hidden = weight.shape
    chunks = b // (_NW * _G)
    idx3d = x.reshape(_NW, chunks, _G * s).astype(jnp.int32)
    return _make_gather(b, s, hidden, vocab)(weight, idx3d)


# speedup vs baseline: 2.0768x; 1.7079x over previous
import functools

import jax
import jax.numpy as jnp
from jax import lax
from jax.experimental import pallas as pl
from jax.experimental.pallas import tpu as pltpu
from jax.experimental.pallas import tpu_sc as plsc

_INFO = plsc.get_sparse_core_info()
_NC = _INFO.num_cores
_NS = _INFO.num_subcores
_NW = _NC * _NS


def _make_gather(n_rows: int, seq: int, hidden: int, vocab: int):
    assert n_rows % (_NW * 8) == 0
    blk = n_rows // _NW
    assert blk <= 128
    assert seq >= 2 and seq % 2 == 0

    mesh = plsc.VectorSubcoreMesh(core_axis_name="c", subcore_axis_name="s")

    @functools.partial(
        pl.kernel,
        out_type=jax.ShapeDtypeStruct((seq, n_rows, hidden), jnp.float32),
        mesh=mesh,
        scratch_types=[
            pltpu.VMEM((seq, blk), jnp.int32),
            pltpu.VMEM((2, blk, hidden), jnp.float32),
            pltpu.SemaphoreType.DMA,
            pltpu.SemaphoreType.DMA,
            pltpu.SemaphoreType.DMA,
            pltpu.SemaphoreType.DMA,
        ],
    )
    def gather_kernel(table_hbm, idx_hbm, out_hbm, idx_v, rows_v, g0, g1, w0, w1):
        gsem = (g0, g1)
        wsem = (w0, w1)
        wid = lax.axis_index("s") * _NC + lax.axis_index("c")
        row_base = wid * blk
        pltpu.sync_copy(
            idx_hbm.at[pl.ds(0, seq), pl.ds(row_base, blk)], idx_v
        )

        def start_gather(j, p):
            pltpu.async_copy(table_hbm.at[idx_v.at[j]], rows_v.at[p], gsem[p])

        def wait_gather(p):
            pltpu.make_async_copy(
                table_hbm.at[idx_v.at[0]], rows_v.at[p], gsem[p]
            ).wait()

        def start_wb(j, p):
            pltpu.async_copy(
                rows_v.at[p], out_hbm.at[j, pl.ds(row_base, blk)], wsem[p]
            )

        def wait_wb(p):
            pltpu.make_async_copy(
                rows_v.at[p], out_hbm.at[0, pl.ds(row_base, blk)], wsem[p]
            ).wait()

        start_gather(0, 0)
        wait_gather(0)
        start_wb(0, 0)
        start_gather(1, 1)

        @pl.loop(1, seq - 1, step=2)
        def _body(jj):
            for b in range(2):
                j = jj + b
                p = (1 + b) % 2
                pn = 1 - p
                wait_gather(p)
                start_wb(j, p)
                wait_wb(pn)
                start_gather(j + 1, pn)

        wait_gather(1)
        start_wb(seq - 1, 1)
        wait_wb(0)
        wait_wb(1)

    return gather_kernel


@jax.jit
def kernel(x, weight):
    b, s = x.shape
    vocab, hidden = weight.shape
    idx_t = x.T.astype(jnp.int32)
    out_t = _make_gather(b, s, hidden, vocab)(weight, idx_t)
    return jnp.transpose(out_t, (1, 0, 2))


# speedup vs baseline: 2.5536x; 1.2296x over previous
import functools

import jax
import jax.numpy as jnp
from jax import lax
from jax.experimental import pallas as pl
from jax.experimental.pallas import tpu as pltpu
from jax.experimental.pallas import tpu_sc as plsc

_INFO = plsc.get_sparse_core_info()
_NC = _INFO.num_cores
_NS = _INFO.num_subcores
_NW = _NC * _NS


def _make_gather(n_rows: int, seq: int, hidden: int, vocab: int):
    assert n_rows % (_NW * 8) == 0
    blk = n_rows // _NW
    assert blk <= 128
    assert seq >= 2 and seq % 2 == 0

    mesh = plsc.VectorSubcoreMesh(core_axis_name="c", subcore_axis_name="s")

    nbuf = 4
    assert (seq - 2) % nbuf == 0

    @functools.partial(
        pl.kernel,
        out_type=jax.ShapeDtypeStruct((seq, n_rows, hidden), jnp.float32),
        mesh=mesh,
        scratch_types=[
            pltpu.VMEM((seq, blk), jnp.int32),
            pltpu.VMEM((nbuf, blk, hidden), jnp.float32),
            [pltpu.SemaphoreType.DMA] * nbuf,
            [pltpu.SemaphoreType.DMA] * nbuf,
        ],
    )
    def gather_kernel(table_hbm, idx_hbm, out_hbm, idx_v, rows_v, gsem, wsem):
        wid = lax.axis_index("s") * _NC + lax.axis_index("c")
        row_base = wid * blk
        pltpu.sync_copy(
            idx_hbm.at[pl.ds(0, seq), pl.ds(row_base, blk)], idx_v
        )

        def start_gather(j, p):
            pltpu.async_copy(table_hbm.at[idx_v.at[j]], rows_v.at[p], gsem[p])

        def wait_gather(p):
            pltpu.make_async_copy(
                table_hbm.at[idx_v.at[0]], rows_v.at[p], gsem[p]
            ).wait()

        def start_wb(j, p):
            pltpu.async_copy(
                rows_v.at[p], out_hbm.at[j, pl.ds(row_base, blk)], wsem[p]
            )

        def wait_wb(p):
            pltpu.make_async_copy(
                rows_v.at[p], out_hbm.at[0, pl.ds(row_base, blk)], wsem[p]
            ).wait()

        def step(j, p):
            wait_gather(p)
            start_wb(j, p)

        def full_step(j, p):
            step(j, p)
            wait_wb((p - 1) % nbuf)
            start_gather(j + nbuf - 1, (p - 1) % nbuf)

        for j in range(nbuf - 1):
            start_gather(j, j)
        step(0, 0)
        start_gather(nbuf - 1, nbuf - 1)

        n_full = seq - nbuf + 1
        n_loop = ((n_full - 1) // nbuf) * nbuf
        @pl.loop(1, 1 + n_loop, step=nbuf)
        def _body(jj):
            for b in range(nbuf):
                full_step(jj + b, (1 + b) % nbuf)
        for j in range(1 + n_loop, n_full):
            full_step(j, j % nbuf)
        for j in range(n_full, seq):
            step(j, j % nbuf)
        for j in range(seq - nbuf, seq):
            wait_wb(j % nbuf)

    return gather_kernel


@jax.jit
def kernel(x, weight):
    b, s = x.shape
    vocab, hidden = weight.shape
    idx_t = x.T.astype(jnp.int32)
    out_t = _make_gather(b, s, hidden, vocab)(weight, idx_t)
    return jnp.transpose(out_t, (1, 0, 2))


# speedup vs baseline: 2.5745x; 1.0082x over previous
import functools

import jax
import jax.numpy as jnp
from jax import lax
from jax.experimental import pallas as pl
from jax.experimental.pallas import tpu as pltpu
from jax.experimental.pallas import tpu_sc as plsc

_INFO = plsc.get_sparse_core_info()
_NC = _INFO.num_cores
_NS = _INFO.num_subcores
_NW = _NC * _NS


def _make_gather(n_rows: int, seq: int, hidden: int, vocab: int):
    assert n_rows % (_NW * 8) == 0
    blk = n_rows // _NW
    assert blk <= 128
    assert seq >= 2 and seq % 2 == 0

    mesh = plsc.VectorSubcoreMesh(core_axis_name="c", subcore_axis_name="s")

    nbuf = 6

    @functools.partial(
        pl.kernel,
        out_type=jax.ShapeDtypeStruct((seq, n_rows, hidden), jnp.float32),
        mesh=mesh,
        scratch_types=[
            pltpu.VMEM((seq, blk), jnp.int32),
            pltpu.VMEM((nbuf, blk, hidden), jnp.float32),
            [pltpu.SemaphoreType.DMA] * nbuf,
            [pltpu.SemaphoreType.DMA] * nbuf,
        ],
    )
    def gather_kernel(table_hbm, idx_hbm, out_hbm, idx_v, rows_v, gsem, wsem):
        wid = lax.axis_index("s") * _NC + lax.axis_index("c")
        row_base = wid * blk
        pltpu.sync_copy(
            idx_hbm.at[pl.ds(0, seq), pl.ds(row_base, blk)], idx_v
        )

        def start_gather(j, p):
            pltpu.async_copy(table_hbm.at[idx_v.at[j]], rows_v.at[p], gsem[p])

        def wait_gather(p):
            pltpu.make_async_copy(
                table_hbm.at[idx_v.at[0]], rows_v.at[p], gsem[p]
            ).wait()

        def start_wb(j, p):
            pltpu.async_copy(
                rows_v.at[p], out_hbm.at[j, pl.ds(row_base, blk)], wsem[p]
            )

        def wait_wb(p):
            pltpu.make_async_copy(
                rows_v.at[p], out_hbm.at[0, pl.ds(row_base, blk)], wsem[p]
            ).wait()

        def step(j, p):
            wait_gather(p)
            start_wb(j, p)

        def full_step(j, p):
            step(j, p)
            wait_wb((p - 1) % nbuf)
            start_gather(j + nbuf - 1, (p - 1) % nbuf)

        for j in range(nbuf - 1):
            start_gather(j, j)
        step(0, 0)
        start_gather(nbuf - 1, nbuf - 1)

        n_full = seq - nbuf + 1
        n_loop = ((n_full - 1) // nbuf) * nbuf
        @pl.loop(1, 1 + n_loop, step=nbuf)
        def _body(jj):
            for b in range(nbuf):
                full_step(jj + b, (1 + b) % nbuf)
        for j in range(1 + n_loop, n_full):
            full_step(j, j % nbuf)
        for j in range(n_full, seq):
            step(j, j % nbuf)
        for j in range(seq - nbuf, seq):
            wait_wb(j % nbuf)

    return gather_kernel


@jax.jit
def kernel(x, weight):
    b, s = x.shape
    vocab, hidden = weight.shape
    idx_t = x.T.astype(jnp.int32)
    out_t = _make_gather(b, s, hidden, vocab)(weight, idx_t)
    return jnp.transpose(out_t, (1, 0, 2))


# speedup vs baseline: 2.5840x; 1.0037x over previous
import functools

import jax
import jax.numpy as jnp
from jax import lax
from jax.experimental import pallas as pl
from jax.experimental.pallas import tpu as pltpu
from jax.experimental.pallas import tpu_sc as plsc

_INFO = plsc.get_sparse_core_info()
_NC = _INFO.num_cores
_NS = _INFO.num_subcores
_NW = _NC * _NS


def _make_gather(n_rows: int, seq: int, hidden: int, vocab: int):
    assert n_rows % (_NW * 8) == 0
    blk = n_rows // _NW
    assert blk <= 128
    assert seq >= 2 and seq % 2 == 0

    mesh = plsc.VectorSubcoreMesh(core_axis_name="c", subcore_axis_name="s")

    nbuf = 7

    @functools.partial(
        pl.kernel,
        out_type=jax.ShapeDtypeStruct((seq, n_rows, hidden), jnp.float32),
        mesh=mesh,
        scratch_types=[
            pltpu.VMEM((seq, blk), jnp.int32),
            pltpu.VMEM((nbuf, blk, hidden), jnp.float32),
            [pltpu.SemaphoreType.DMA] * nbuf,
            [pltpu.SemaphoreType.DMA] * nbuf,
        ],
    )
    def gather_kernel(table_hbm, idx_hbm, out_hbm, idx_v, rows_v, gsem, wsem):
        wid = lax.axis_index("s") * _NC + lax.axis_index("c")
        row_base = wid * blk
        pltpu.sync_copy(
            idx_hbm.at[pl.ds(0, seq), pl.ds(row_base, blk)], idx_v
        )

        def start_gather(j, p):
            pltpu.async_copy(table_hbm.at[idx_v.at[j]], rows_v.at[p], gsem[p])

        def wait_gather(p):
            pltpu.make_async_copy(
                table_hbm.at[idx_v.at[0]], rows_v.at[p], gsem[p]
            ).wait()

        def start_wb(j, p):
            pltpu.async_copy(
                rows_v.at[p], out_hbm.at[j, pl.ds(row_base, blk)], wsem[p]
            )

        def wait_wb(p):
            pltpu.make_async_copy(
                rows_v.at[p], out_hbm.at[0, pl.ds(row_base, blk)], wsem[p]
            ).wait()

        def step(j, p):
            wait_gather(p)
            start_wb(j, p)

        def full_step(j, p):
            step(j, p)
            wait_wb((p - 1) % nbuf)
            start_gather(j + nbuf - 1, (p - 1) % nbuf)

        for j in range(nbuf - 1):
            start_gather(j, j)
        step(0, 0)
        start_gather(nbuf - 1, nbuf - 1)

        n_full = seq - nbuf + 1
        n_loop = ((n_full - 1) // nbuf) * nbuf
        @pl.loop(1, 1 + n_loop, step=nbuf)
        def _body(jj):
            for b in range(nbuf):
                full_step(jj + b, (1 + b) % nbuf)
        for j in range(1 + n_loop, n_full):
            full_step(j, j % nbuf)
        for j in range(n_full, seq):
            step(j, j % nbuf)
        for j in range(seq - nbuf, seq):
            wait_wb(j % nbuf)

    return gather_kernel


@jax.jit
def kernel(x, weight):
    b, s = x.shape
    vocab, hidden = weight.shape
    idx_t = x.T.astype(jnp.int32)
    out_t = _make_gather(b, s, hidden, vocab)(weight, idx_t)
    return jnp.transpose(out_t, (1, 0, 2))
